# baseline (speedup 1.0000x reference)
import functools

import jax
import jax.numpy as jnp
from jax import lax
from jax.experimental import pallas as pl
from jax.experimental.pallas import tpu as pltpu
from jax.experimental.pallas import tpu_sc as plsc

NC = 2
NS = 16
NW = NC * NS
CHUNK = 80


def _f32(v):
    return v.astype(jnp.float32)


def _bf(v):
    return v.astype(jnp.bfloat16)


def _dot(a, b):
    return jnp.dot(_bf(a), _bf(b), preferred_element_type=jnp.float32)


def _pre_body(x_ref, wi_ref, wj_ref, b0_ref, pi_ref, pj_ref):
    xb = x_ref[...]
    pi_ref[...] = _dot(xb, wi_ref[...]) + b0_ref[...]
    pj_ref[...] = _dot(xb, wj_ref[...])


def _precompute(x, w_i, w_j, b0, block):
    n, d = x.shape
    h = w_i.shape[1]
    grid = n // block
    return pl.pallas_call(
        _pre_body,
        grid=(grid,),
        in_specs=[
            pl.BlockSpec((block, d), lambda i: (i, 0)),
            pl.BlockSpec((d, h), lambda i: (0, 0)),
            pl.BlockSpec((d, h), lambda i: (0, 0)),
            pl.BlockSpec((1, h), lambda i: (0, 0)),
        ],
        out_specs=[
            pl.BlockSpec((block, h), lambda i: (i, 0)),
            pl.BlockSpec((block, h), lambda i: (i, 0)),
        ],
        out_shape=[
            jax.ShapeDtypeStruct((n, h), jnp.float32),
            jax.ShapeDtypeStruct((n, h), jnp.float32),
        ],
    )(x, w_i, w_j, b0)


def _gather_add(pi, pj, dst3, src3):
    n, h = pi.shape
    nw, chunks, c = dst3.shape
    assert nw == NW and c == CHUNK
    assert chunks % 2 == 1 and chunks >= 7
    e = nw * chunks * c
    per_w = chunks * c
    mesh = plsc.VectorSubcoreMesh(core_axis_name="c", subcore_axis_name="s")

    @functools.partial(
        pl.kernel,
        out_type=jax.ShapeDtypeStruct((e, h), jnp.float32),
        mesh=mesh,
        scratch_types=[
            pltpu.VMEM((chunks, c), jnp.int32),
            pltpu.VMEM((chunks, c), jnp.int32),
            pltpu.VMEM((2, c, h), jnp.float32),
            pltpu.VMEM((2, c, h), jnp.float32),
            pltpu.VMEM((2, c, h), jnp.float32),
            pltpu.SemaphoreType.DMA,
            pltpu.SemaphoreType.DMA,
            pltpu.SemaphoreType.DMA,
            pltpu.SemaphoreType.DMA,
            pltpu.SemaphoreType.DMA,
        ],
    )
    def k(pi_hbm, pj_hbm, dst_hbm, src_hbm, g_hbm, dst_sl, src_sl,
          a_v, b_v, s_v, gsem0, gsem1, ssem0, ssem1, isem):
        wid = lax.axis_index("s") * NC + lax.axis_index("c")
        base0 = wid * per_w
        gsems = (gsem0, gsem1)
        ssems = (ssem0, ssem1)

        cpi = pltpu.async_copy(dst_hbm.at[wid], dst_sl, isem)
        cps = pltpu.async_copy(src_hbm.at[wid], src_sl, isem)
        cpi.wait()
        cps.wait()

        def issue_gather(kk, si):
            pltpu.async_copy(pi_hbm.at[dst_sl.at[kk]], a_v.at[si], gsems[si])
            pltpu.async_copy(pj_hbm.at[src_sl.at[kk]], b_v.at[si], gsems[si])

        def do_chunk(kk, si, wait_store, issue_next):
            a_r = a_v.at[si]
            b_r = b_v.at[si]
            s_r = s_v.at[si]
            base = base0 + kk * c
            out_slice = g_hbm.at[pl.ds(base, c)]
            pltpu.make_async_copy(pi_hbm.at[dst_sl.at[kk]], a_r,
                                  gsems[si]).wait()
            pltpu.make_async_copy(pj_hbm.at[src_sl.at[kk]], b_r,
                                  gsems[si]).wait()
            if wait_store:
                pltpu.make_async_copy(s_r, out_slice, ssems[si]).wait()

            @plsc.parallel_loop(0, c)
            def _row(i):
                for j in range(h // 16):
                    sl = pl.ds(j * 16, 16)
                    s_r[i, sl] = a_r[i, sl] + b_r[i, sl]

            pltpu.async_copy(s_r, out_slice, ssems[si])
            if issue_next:
                issue_gather(kk + 2, si)

        issue_gather(0, 0)
        issue_gather(1, 1)
        do_chunk(0, 0, False, True)
        do_chunk(1, 1, False, True)

        @pl.loop(1, (chunks - 5) // 2 + 1)
        def _pair(t):
            do_chunk(2 * t, 0, True, True)
            do_chunk(2 * t + 1, 1, True, True)

        do_chunk(chunks - 3, 0, True, True)
        do_chunk(chunks - 2, 1, True, False)
        do_chunk(chunks - 1, 0, True, False)
        pltpu.make_async_copy(
            s_v.at[0], g_hbm.at[pl.ds(base0, c)], ssem0).wait()
        pltpu.make_async_copy(
            s_v.at[1], g_hbm.at[pl.ds(base0, c)], ssem1).wait()

    return k(pi, pj, dst3, src3)


def _edge_body(g_ref, ea_ref, w0e_ref, w1_ref, b1_ref, lng_ref, lnb_ref,
               out_ref):
    ea = ea_ref[...]
    hh = jnp.maximum(g_ref[...] + _dot(ea, w0e_ref[...]), 0.0)
    hh = _dot(hh, w1_ref[...]) + b1_ref[...]
    m = jnp.mean(hh, axis=1, keepdims=True)
    v = jnp.mean((hh - m) ** 2, axis=1, keepdims=True)
    msg = (hh - m) * lax.rsqrt(v + 1e-5) * lng_ref[...] + lnb_ref[...]
    out_ref[...] = ea + msg


def _edge_mlp(g, ea, w0e, w1, b1, lng, lnb, block):
    e, d = ea.shape
    h = w0e.shape[1]
    grid = e // block
    return pl.pallas_call(
        _edge_body,
        grid=(grid,),
        in_specs=[
            pl.BlockSpec((block, h), lambda i: (i, 0)),
            pl.BlockSpec((block, d), lambda i: (i, 0)),
            pl.BlockSpec((d, h), lambda i: (0, 0)),
            pl.BlockSpec((h, d), lambda i: (0, 0)),
            pl.BlockSpec((1, d), lambda i: (0, 0)),
            pl.BlockSpec((1, d), lambda i: (0, 0)),
            pl.BlockSpec((1, d), lambda i: (0, 0)),
        ],
        out_specs=pl.BlockSpec((block, d), lambda i: (i, 0)),
        out_shape=jax.ShapeDtypeStruct((e, d), jnp.float32),
    )(g, ea, w0e, w1, b1, lng, lnb)


def _scatter_add(ne, dst3, n):
    e, h = ne.shape
    nw, chunks, c = dst3.shape
    assert nw == NW and c == CHUNK
    assert chunks % 2 == 1 and chunks >= 7
    per_w = chunks * c
    n_pad = ((n + c * NS - 1) // (c * NS)) * (c * NS)
    rows_per_tile = n_pad // NS
    zreps = rows_per_tile // c
    mesh = plsc.VectorSubcoreMesh(core_axis_name="c", subcore_axis_name="s")

    @functools.partial(
        pl.kernel,
        out_type=jax.ShapeDtypeStruct((NC, n_pad, h), jnp.float32),
        mesh=mesh,
        scratch_types=[
            pltpu.VMEM((chunks, c), jnp.int32),
            pltpu.VMEM((2, c, h), jnp.float32),
            pltpu.VMEM_SHARED((n_pad, h), jnp.float32),
            pltpu.SemaphoreType.DMA,
            pltpu.SemaphoreType.DMA,
            pltpu.SemaphoreType.DMA,
        ],
    )
    def k(ne_hbm, dst_hbm, out_hbm, idx_sl, u_v, acc_sh,
          usem0, usem1, isem):
        cc = lax.axis_index("c")
        ss = lax.axis_index("s")
        wid = ss * NC + cc
        base0 = wid * per_w
        usems = (usem0, usem1)

        cpi = pltpu.async_copy(dst_hbm.at[wid], idx_sl, isem)
        z_v = u_v.at[0]

        @pl.loop(0, c)
        def _zrow(i):
            for j in range(h // 16):
                z_v[i, pl.ds(j * 16, 16)] = jnp.zeros((16,), jnp.float32)

        for r in range(zreps):
            pltpu.sync_copy(
                z_v, acc_sh.at[pl.ds(ss * rows_per_tile + r * c, c)]
            )
        cpi.wait()
        plsc.subcore_barrier()

        def issue_load(kk, si):
            pltpu.async_copy(
                ne_hbm.at[pl.ds(base0 + kk * c, c)], u_v.at[si], usems[si])

        def do_chunk(kk, si, issue_next):
            u_r = u_v.at[si]
            pltpu.make_async_copy(
                ne_hbm.at[pl.ds(base0, c)], u_r, usems[si]).wait()
            pltpu.sync_copy(u_r, acc_sh.at[idx_sl.at[kk]], add=True)
            if issue_next:
                issue_load(kk + 2, si)

        issue_load(0, 0)
        issue_load(1, 1)
        do_chunk(0, 0, True)
        do_chunk(1, 1, True)

        @pl.loop(1, (chunks - 5) // 2 + 1)
        def _pair(t):
            do_chunk(2 * t, 0, True)
            do_chunk(2 * t + 1, 1, True)

        do_chunk(chunks - 3, 0, True)
        do_chunk(chunks - 2, 1, False)
        do_chunk(chunks - 1, 0, False)

        plsc.subcore_barrier()
        pltpu.sync_copy(
            acc_sh.at[pl.ds(ss * rows_per_tile, rows_per_tile)],
            out_hbm.at[cc, pl.ds(ss * rows_per_tile, rows_per_tile)],
        )

    return k(ne, dst3)


def _node_body(x_ref, p0_ref, p1_ref, ux_ref, ua_ref, b0_ref, w1_ref, b1_ref,
               lng_ref, lnb_ref, out_ref):
    xb = x_ref[...]
    agg = p0_ref[...] + p1_ref[...]
    u = jnp.maximum(
        _dot(xb, ux_ref[...]) + _dot(agg, ua_ref[...]) + b0_ref[...], 0.0)
    u = _dot(u, w1_ref[...]) + b1_ref[...]
    m = jnp.mean(u, axis=1, keepdims=True)
    v = jnp.mean((u - m) ** 2, axis=1, keepdims=True)
    out_ref[...] = (
        xb + (u - m) * lax.rsqrt(v + 1e-5) * lng_ref[...] + lnb_ref[...]
    )


def _node_mlp(x, p0, p1, ux, ua, b0, w1, b1, lng, lnb, block):
    n, d = x.shape
    h = ux.shape[1]
    grid = n // block
    return pl.pallas_call(
        _node_body,
        grid=(grid,),
        in_specs=[
            pl.BlockSpec((block, d), lambda i: (i, 0)),
            pl.BlockSpec((block, d), lambda i: (i, 0)),
            pl.BlockSpec((block, d), lambda i: (i, 0)),
            pl.BlockSpec((d, h), lambda i: (0, 0)),
            pl.BlockSpec((d, h), lambda i: (0, 0)),
            pl.BlockSpec((1, h), lambda i: (0, 0)),
            pl.BlockSpec((h, d), lambda i: (0, 0)),
            pl.BlockSpec((1, d), lambda i: (0, 0)),
            pl.BlockSpec((1, d), lambda i: (0, 0)),
            pl.BlockSpec((1, d), lambda i: (0, 0)),
        ],
        out_specs=pl.BlockSpec((block, d), lambda i: (i, 0)),
        out_shape=jax.ShapeDtypeStruct((n, d), jnp.float32),
    )(x, p0, p1, ux, ua, b0, w1, b1, lng, lnb)


def kernel(x, edge_index, edge_attr, msg_W0, msg_b0, msg_W1, msg_b1,
           msg_lng, msg_lnb, upd_W0, upd_b0, upd_W1, upd_b1, upd_lng,
           upd_lnb):
    n, d = x.shape
    e = edge_index.shape[1]
    h = msg_W0.shape[1]
    chunks = e // (NW * CHUNK)
    assert chunks * NW * CHUNK == e
    src3 = edge_index[0].reshape(NW, chunks, CHUNK)
    dst3 = edge_index[1].reshape(NW, chunks, CHUNK)

    w0_i = msg_W0[:d]
    w0_j = msg_W0[d:2 * d]
    w0_e = msg_W0[2 * d:]
    b0 = msg_b0.reshape(1, h)
    b1 = msg_b1.reshape(1, d)
    lng = msg_lng.reshape(1, d)
    lnb = msg_lnb.reshape(1, d)

    pi, pj = _precompute(x, w0_i, w0_j, b0, block=1000)
    g = _gather_add(pi, pj, dst3, src3)
    new_edge_attr = _edge_mlp(g, edge_attr, w0_e, msg_W1, b1, lng, lnb,
                              block=10000)
    partials = _scatter_add(new_edge_attr, dst3, n)

    ux = upd_W0[:d]
    ua = upd_W0[d:]
    new_x = _node_mlp(
        x, partials[0], partials[1], ux, ua, upd_b0.reshape(1, h),
        upd_W1, upd_b1.reshape(1, d), upd_lng.reshape(1, d),
        upd_lnb.reshape(1, d), block=1000,
    )
    return (new_x, new_edge_attr)


# speedup vs baseline: 1.0099x; 1.0099x over previous
import functools

import jax
import jax.numpy as jnp
from jax import lax
from jax.experimental import pallas as pl
from jax.experimental.pallas import tpu as pltpu
from jax.experimental.pallas import tpu_sc as plsc

NC = 2
NS = 16
NW = NC * NS
CHUNK = 80


def _f32(v):
    return v.astype(jnp.float32)


def _bf(v):
    return v.astype(jnp.bfloat16)


def _dot(a, b):
    return jnp.dot(_bf(a), _bf(b), preferred_element_type=jnp.float32)


def _pre_body(x_ref, wi_ref, wj_ref, b0_ref, pi_ref, pj_ref):
    xb = x_ref[...]
    pi_ref[...] = _dot(xb, wi_ref[...]) + b0_ref[...]
    pj_ref[...] = _dot(xb, wj_ref[...])


def _precompute(x, w_i, w_j, b0, block):
    n, d = x.shape
    h = w_i.shape[1]
    grid = n // block
    return pl.pallas_call(
        _pre_body,
        grid=(grid,),
        in_specs=[
            pl.BlockSpec((block, d), lambda i: (i, 0)),
            pl.BlockSpec((d, h), lambda i: (0, 0)),
            pl.BlockSpec((d, h), lambda i: (0, 0)),
            pl.BlockSpec((1, h), lambda i: (0, 0)),
        ],
        out_specs=[
            pl.BlockSpec((block, h), lambda i: (i, 0)),
            pl.BlockSpec((block, h), lambda i: (i, 0)),
        ],
        out_shape=[
            jax.ShapeDtypeStruct((n, h), jnp.float32),
            jax.ShapeDtypeStruct((n, h), jnp.float32),
        ],
    )(x, w_i, w_j, b0)


def _gather_add(pi, pj, dst3, src3):
    n, h = pi.shape
    nw, chunks, c = dst3.shape
    assert nw == NW and c == CHUNK
    assert chunks % 2 == 1 and chunks >= 7
    e = nw * chunks * c
    per_w = chunks * c
    mesh = plsc.VectorSubcoreMesh(core_axis_name="c", subcore_axis_name="s")

    @functools.partial(
        pl.kernel,
        out_type=jax.ShapeDtypeStruct((e, h), jnp.float32),
        mesh=mesh,
        scratch_types=[
            pltpu.VMEM((chunks, c), jnp.int32),
            pltpu.VMEM((chunks, c), jnp.int32),
            pltpu.VMEM((2, c, h), jnp.float32),
            pltpu.VMEM((2, c, h), jnp.float32),
            pltpu.VMEM((2, c, h), jnp.float32),
            pltpu.SemaphoreType.DMA,
            pltpu.SemaphoreType.DMA,
            pltpu.SemaphoreType.DMA,
            pltpu.SemaphoreType.DMA,
            pltpu.SemaphoreType.DMA,
        ],
    )
    def k(pi_hbm, pj_hbm, dst_hbm, src_hbm, g_hbm, dst_sl, src_sl,
          a_v, b_v, s_v, gsem0, gsem1, ssem0, ssem1, isem):
        wid = lax.axis_index("s") * NC + lax.axis_index("c")
        base0 = wid * per_w
        gsems = (gsem0, gsem1)
        ssems = (ssem0, ssem1)

        cpi = pltpu.async_copy(dst_hbm.at[wid], dst_sl, isem)
        cps = pltpu.async_copy(src_hbm.at[wid], src_sl, isem)
        cpi.wait()
        cps.wait()

        def issue_gather(kk, si):
            pltpu.async_copy(pi_hbm.at[dst_sl.at[kk]], a_v.at[si], gsems[si])
            pltpu.async_copy(pj_hbm.at[src_sl.at[kk]], b_v.at[si], gsems[si])

        def do_chunk(kk, si, wait_store, issue_next):
            a_r = a_v.at[si]
            b_r = b_v.at[si]
            s_r = s_v.at[si]
            base = base0 + kk * c
            out_slice = g_hbm.at[pl.ds(base, c)]
            pltpu.make_async_copy(pi_hbm.at[dst_sl.at[kk]], a_r,
                                  gsems[si]).wait()
            pltpu.make_async_copy(pj_hbm.at[src_sl.at[kk]], b_r,
                                  gsems[si]).wait()
            if wait_store:
                pltpu.make_async_copy(s_r, out_slice, ssems[si]).wait()

            @plsc.parallel_loop(0, c)
            def _row(i):
                for j in range(h // 16):
                    sl = pl.ds(j * 16, 16)
                    s_r[i, sl] = a_r[i, sl] + b_r[i, sl]

            pltpu.async_copy(s_r, out_slice, ssems[si])
            if issue_next:
                issue_gather(kk + 2, si)

        issue_gather(0, 0)
        issue_gather(1, 1)
        do_chunk(0, 0, False, True)
        do_chunk(1, 1, False, True)

        @pl.loop(1, (chunks - 5) // 2 + 1)
        def _pair(t):
            do_chunk(2 * t, 0, True, True)
            do_chunk(2 * t + 1, 1, True, True)

        do_chunk(chunks - 3, 0, True, True)
        do_chunk(chunks - 2, 1, True, False)
        do_chunk(chunks - 1, 0, True, False)
        pltpu.make_async_copy(
            s_v.at[0], g_hbm.at[pl.ds(base0, c)], ssem0).wait()
        pltpu.make_async_copy(
            s_v.at[1], g_hbm.at[pl.ds(base0, c)], ssem1).wait()

    return k(pi, pj, dst3, src3)


def _edge_body(g_ref, ea_ref, w0e_ref, w1_ref, b1_ref, lng_ref, lnb_ref,
               out_ref):
    ea = ea_ref[...]
    hh = jnp.maximum(g_ref[...] + _dot(ea, w0e_ref[...]), 0.0)
    hh = _dot(hh, w1_ref[...]) + b1_ref[...]
    m = jnp.mean(hh, axis=1, keepdims=True)
    v = jnp.mean((hh - m) ** 2, axis=1, keepdims=True)
    msg = (hh - m) * lax.rsqrt(v + 1e-5) * lng_ref[...] + lnb_ref[...]
    out_ref[...] = ea + msg


def _edge_mlp(g, ea, w0e, w1, b1, lng, lnb, block):
    e, d = ea.shape
    h = w0e.shape[1]
    grid = e // block
    return pl.pallas_call(
        _edge_body,
        grid=(grid,),
        in_specs=[
            pl.BlockSpec((block, h), lambda i: (i, 0)),
            pl.BlockSpec((block, d), lambda i: (i, 0)),
            pl.BlockSpec((d, h), lambda i: (0, 0)),
            pl.BlockSpec((h, d), lambda i: (0, 0)),
            pl.BlockSpec((1, d), lambda i: (0, 0)),
            pl.BlockSpec((1, d), lambda i: (0, 0)),
            pl.BlockSpec((1, d), lambda i: (0, 0)),
        ],
        out_specs=pl.BlockSpec((block, d), lambda i: (i, 0)),
        out_shape=jax.ShapeDtypeStruct((e, d), jnp.float32),
    )(g, ea, w0e, w1, b1, lng, lnb)


def _scatter_add(ne, dst3, n):
    e, h = ne.shape
    nw, chunks, c = dst3.shape
    assert nw == NW and c == CHUNK
    assert chunks % 2 == 1 and chunks >= 7
    per_w = chunks * c
    n_pad = ((n + c * NS - 1) // (c * NS)) * (c * NS)
    rows_per_tile = n_pad // NS
    zreps = rows_per_tile // c
    mesh = plsc.VectorSubcoreMesh(core_axis_name="c", subcore_axis_name="s")

    @functools.partial(
        pl.kernel,
        out_type=jax.ShapeDtypeStruct((NC, n_pad, h), jnp.float32),
        mesh=mesh,
        scratch_types=[
            pltpu.VMEM((chunks, c), jnp.int32),
            pltpu.VMEM((2, c, h), jnp.float32),
            pltpu.VMEM_SHARED((n_pad, h), jnp.float32),
            pltpu.SemaphoreType.DMA,
            pltpu.SemaphoreType.DMA,
            pltpu.SemaphoreType.DMA,
        ],
    )
    def k(ne_hbm, dst_hbm, out_hbm, idx_sl, u_v, acc_sh,
          usem0, usem1, isem):
        cc = lax.axis_index("c")
        ss = lax.axis_index("s")
        wid = ss * NC + cc
        base0 = wid * per_w
        usems = (usem0, usem1)

        cpi = pltpu.async_copy(dst_hbm.at[wid], idx_sl, isem)
        z_v = u_v.at[0]

        @pl.loop(0, c)
        def _zrow(i):
            for j in range(h // 16):
                z_v[i, pl.ds(j * 16, 16)] = jnp.zeros((16,), jnp.float32)

        for r in range(zreps):
            pltpu.sync_copy(
                z_v, acc_sh.at[pl.ds(ss * rows_per_tile + r * c, c)]
            )
        cpi.wait()
        plsc.subcore_barrier()

        def issue_load(kk, si):
            pltpu.async_copy(
                ne_hbm.at[pl.ds(base0 + kk * c, c)], u_v.at[si], usems[si])

        def do_chunk(kk, si, issue_next):
            u_r = u_v.at[si]
            pltpu.make_async_copy(
                ne_hbm.at[pl.ds(base0, c)], u_r, usems[si]).wait()
            pltpu.sync_copy(u_r, acc_sh.at[idx_sl.at[kk]], add=True)
            if issue_next:
                issue_load(kk + 2, si)

        issue_load(0, 0)
        issue_load(1, 1)
        do_chunk(0, 0, True)
        do_chunk(1, 1, True)

        @pl.loop(1, (chunks - 5) // 2 + 1)
        def _pair(t):
            do_chunk(2 * t, 0, True)
            do_chunk(2 * t + 1, 1, True)

        do_chunk(chunks - 3, 0, True)
        do_chunk(chunks - 2, 1, False)
        do_chunk(chunks - 1, 0, False)

        plsc.subcore_barrier()
        pltpu.sync_copy(
            acc_sh.at[pl.ds(ss * rows_per_tile, rows_per_tile)],
            out_hbm.at[cc, pl.ds(ss * rows_per_tile, rows_per_tile)],
        )

    return k(ne, dst3)


def _node_body(x_ref, p0_ref, p1_ref, ux_ref, ua_ref, b0_ref, w1_ref, b1_ref,
               lng_ref, lnb_ref, out_ref):
    xb = x_ref[...]
    agg = p0_ref[...] + p1_ref[...]
    u = jnp.maximum(
        _dot(xb, ux_ref[...]) + _dot(agg, ua_ref[...]) + b0_ref[...], 0.0)
    u = _dot(u, w1_ref[...]) + b1_ref[...]
    m = jnp.mean(u, axis=1, keepdims=True)
    v = jnp.mean((u - m) ** 2, axis=1, keepdims=True)
    out_ref[...] = (
        xb + (u - m) * lax.rsqrt(v + 1e-5) * lng_ref[...] + lnb_ref[...]
    )


def _node_mlp(x, p0, p1, ux, ua, b0, w1, b1, lng, lnb, block):
    n, d = x.shape
    h = ux.shape[1]
    grid = n // block
    return pl.pallas_call(
        _node_body,
        grid=(grid,),
        in_specs=[
            pl.BlockSpec((block, d), lambda i: (i, 0)),
            pl.BlockSpec((block, d), lambda i: (i, 0)),
            pl.BlockSpec((block, d), lambda i: (i, 0)),
            pl.BlockSpec((d, h), lambda i: (0, 0)),
            pl.BlockSpec((d, h), lambda i: (0, 0)),
            pl.BlockSpec((1, h), lambda i: (0, 0)),
            pl.BlockSpec((h, d), lambda i: (0, 0)),
            pl.BlockSpec((1, d), lambda i: (0, 0)),
            pl.BlockSpec((1, d), lambda i: (0, 0)),
            pl.BlockSpec((1, d), lambda i: (0, 0)),
        ],
        out_specs=pl.BlockSpec((block, d), lambda i: (i, 0)),
        out_shape=jax.ShapeDtypeStruct((n, d), jnp.float32),
    )(x, p0, p1, ux, ua, b0, w1, b1, lng, lnb)


def kernel(x, edge_index, edge_attr, msg_W0, msg_b0, msg_W1, msg_b1,
           msg_lng, msg_lnb, upd_W0, upd_b0, upd_W1, upd_b1, upd_lng,
           upd_lnb):
    n, d = x.shape
    e = edge_index.shape[1]
    h = msg_W0.shape[1]
    chunks = e // (NW * CHUNK)
    assert chunks * NW * CHUNK == e
    src3 = edge_index[0].reshape(NW, chunks, CHUNK)
    dst3 = edge_index[1].reshape(NW, chunks, CHUNK)

    w0_i = msg_W0[:d]
    w0_j = msg_W0[d:2 * d]
    w0_e = msg_W0[2 * d:]
    b0 = msg_b0.reshape(1, h)
    b1 = msg_b1.reshape(1, d)
    lng = msg_lng.reshape(1, d)
    lnb = msg_lnb.reshape(1, d)

    pi, pj = _precompute(x, w0_i, w0_j, b0, block=1000)
    g = _gather_add(pi, pj, dst3, src3)
    new_edge_attr = _edge_mlp(g, edge_attr, w0_e, msg_W1, b1, lng, lnb,
                              block=8000)
    partials = _scatter_add(new_edge_attr, dst3, n)

    ux = upd_W0[:d]
    ua = upd_W0[d:]
    new_x = _node_mlp(
        x, partials[0], partials[1], ux, ua, upd_b0.reshape(1, h),
        upd_W1, upd_b1.reshape(1, d), upd_lng.reshape(1, d),
        upd_lnb.reshape(1, d), block=1000,
    )
    return (new_x, new_edge_attr)


# speedup vs baseline: 1.0169x; 1.0070x over previous
import functools

import jax
import jax.numpy as jnp
from jax import lax
from jax.experimental import pallas as pl
from jax.experimental.pallas import tpu as pltpu
from jax.experimental.pallas import tpu_sc as plsc

NC = 2
NS = 16
NW = NC * NS
CHUNK = 80


def _f32(v):
    return v.astype(jnp.float32)


def _bf(v):
    return v.astype(jnp.bfloat16)


def _dot(a, b):
    return jnp.dot(_bf(a), _bf(b), preferred_element_type=jnp.float32)


def _pre_body(x_ref, wi_ref, wj_ref, b0_ref, pi_ref, pj_ref):
    xb = x_ref[...]
    pi_ref[...] = _dot(xb, wi_ref[...]) + b0_ref[...]
    pj_ref[...] = _dot(xb, wj_ref[...])


def _precompute(x, w_i, w_j, b0, block):
    n, d = x.shape
    h = w_i.shape[1]
    grid = n // block
    return pl.pallas_call(
        _pre_body,
        grid=(grid,),
        in_specs=[
            pl.BlockSpec((block, d), lambda i: (i, 0)),
            pl.BlockSpec((d, h), lambda i: (0, 0)),
            pl.BlockSpec((d, h), lambda i: (0, 0)),
            pl.BlockSpec((1, h), lambda i: (0, 0)),
        ],
        out_specs=[
            pl.BlockSpec((block, h), lambda i: (i, 0)),
            pl.BlockSpec((block, h), lambda i: (i, 0)),
        ],
        out_shape=[
            jax.ShapeDtypeStruct((n, h), jnp.float32),
            jax.ShapeDtypeStruct((n, h), jnp.float32),
        ],
    )(x, w_i, w_j, b0)


def _gather_add(pi, pj, dst3, src3):
    n, h = pi.shape
    nw, chunks, c = dst3.shape
    assert nw == NW and c == CHUNK
    assert chunks % 2 == 1 and chunks >= 7
    e = nw * chunks * c
    per_w = chunks * c
    mesh = plsc.VectorSubcoreMesh(core_axis_name="c", subcore_axis_name="s")

    @functools.partial(
        pl.kernel,
        out_type=jax.ShapeDtypeStruct((e, h), jnp.float32),
        mesh=mesh,
        scratch_types=[
            pltpu.VMEM((chunks, c), jnp.int32),
            pltpu.VMEM((chunks, c), jnp.int32),
            pltpu.VMEM((2, c, h), jnp.float32),
            pltpu.VMEM((2, c, h), jnp.float32),
            pltpu.VMEM((2, c, h), jnp.float32),
            pltpu.SemaphoreType.DMA,
            pltpu.SemaphoreType.DMA,
            pltpu.SemaphoreType.DMA,
            pltpu.SemaphoreType.DMA,
            pltpu.SemaphoreType.DMA,
        ],
    )
    def k(pi_hbm, pj_hbm, dst_hbm, src_hbm, g_hbm, dst_sl, src_sl,
          a_v, b_v, s_v, gsem0, gsem1, ssem0, ssem1, isem):
        wid = lax.axis_index("s") * NC + lax.axis_index("c")
        base0 = wid * per_w
        gsems = (gsem0, gsem1)
        ssems = (ssem0, ssem1)

        cpi = pltpu.async_copy(dst_hbm.at[wid], dst_sl, isem)
        cps = pltpu.async_copy(src_hbm.at[wid], src_sl, isem)
        cpi.wait()
        cps.wait()

        def issue_gather(kk, si):
            pltpu.async_copy(pi_hbm.at[dst_sl.at[kk]], a_v.at[si], gsems[si])
            pltpu.async_copy(pj_hbm.at[src_sl.at[kk]], b_v.at[si], gsems[si])

        def do_chunk(kk, si, wait_store, issue_next):
            a_r = a_v.at[si]
            b_r = b_v.at[si]
            s_r = s_v.at[si]
            base = base0 + kk * c
            out_slice = g_hbm.at[pl.ds(base, c)]
            pltpu.make_async_copy(pi_hbm.at[dst_sl.at[kk]], a_r,
                                  gsems[si]).wait()
            pltpu.make_async_copy(pj_hbm.at[src_sl.at[kk]], b_r,
                                  gsems[si]).wait()
            if wait_store:
                pltpu.make_async_copy(s_r, out_slice, ssems[si]).wait()

            @plsc.parallel_loop(0, c)
            def _row(i):
                for j in range(h // 16):
                    sl = pl.ds(j * 16, 16)
                    s_r[i, sl] = a_r[i, sl] + b_r[i, sl]

            pltpu.async_copy(s_r, out_slice, ssems[si])
            if issue_next:
                issue_gather(kk + 2, si)

        issue_gather(0, 0)
        issue_gather(1, 1)
        do_chunk(0, 0, False, True)
        do_chunk(1, 1, False, True)

        @pl.loop(1, (chunks - 5) // 2 + 1)
        def _pair(t):
            do_chunk(2 * t, 0, True, True)
            do_chunk(2 * t + 1, 1, True, True)

        do_chunk(chunks - 3, 0, True, True)
        do_chunk(chunks - 2, 1, True, False)
        do_chunk(chunks - 1, 0, True, False)
        pltpu.make_async_copy(
            s_v.at[0], g_hbm.at[pl.ds(base0, c)], ssem0).wait()
        pltpu.make_async_copy(
            s_v.at[1], g_hbm.at[pl.ds(base0, c)], ssem1).wait()

    return k(pi, pj, dst3, src3)


def _edge_body(g_ref, ea_ref, w0e_ref, w1_ref, b1_ref, lng_ref, lnb_ref,
               out_ref):
    ea = ea_ref[...]
    hh = jnp.maximum(g_ref[...] + _dot(ea, w0e_ref[...]), 0.0)
    hh = _dot(hh, w1_ref[...]) + b1_ref[...]
    m = jnp.mean(hh, axis=1, keepdims=True)
    v = jnp.mean((hh - m) ** 2, axis=1, keepdims=True)
    msg = (hh - m) * lax.rsqrt(v + 1e-5) * lng_ref[...] + lnb_ref[...]
    out_ref[...] = ea + msg


def _edge_mlp(g, ea, w0e, w1, b1, lng, lnb, block):
    e, d = ea.shape
    h = w0e.shape[1]
    grid = e // block
    return pl.pallas_call(
        _edge_body,
        grid=(grid,),
        in_specs=[
            pl.BlockSpec((block, h), lambda i: (i, 0)),
            pl.BlockSpec((block, d), lambda i: (i, 0)),
            pl.BlockSpec((d, h), lambda i: (0, 0)),
            pl.BlockSpec((h, d), lambda i: (0, 0)),
            pl.BlockSpec((1, d), lambda i: (0, 0)),
            pl.BlockSpec((1, d), lambda i: (0, 0)),
            pl.BlockSpec((1, d), lambda i: (0, 0)),
        ],
        out_specs=pl.BlockSpec((block, d), lambda i: (i, 0)),
        out_shape=jax.ShapeDtypeStruct((e, d), jnp.float32),
    )(g, ea, w0e, w1, b1, lng, lnb)


def _scatter_add(ne, dst3, n):
    e, h = ne.shape
    nw, chunks, c = dst3.shape
    assert nw == NW and c == CHUNK
    assert chunks % 2 == 1 and chunks >= 7
    per_w = chunks * c
    n_pad = ((n + c * NS - 1) // (c * NS)) * (c * NS)
    rows_per_tile = n_pad // NS
    zreps = rows_per_tile // c
    mesh = plsc.VectorSubcoreMesh(core_axis_name="c", subcore_axis_name="s")

    @functools.partial(
        pl.kernel,
        out_type=jax.ShapeDtypeStruct((NC, n_pad, h), jnp.float32),
        mesh=mesh,
        scratch_types=[
            pltpu.VMEM((chunks, c), jnp.int32),
            pltpu.VMEM((2, c, h), jnp.float32),
            pltpu.VMEM_SHARED((n_pad, h), jnp.float32),
            pltpu.SemaphoreType.DMA,
            pltpu.SemaphoreType.DMA,
            pltpu.SemaphoreType.DMA,
        ],
    )
    def k(ne_hbm, dst_hbm, out_hbm, idx_sl, u_v, acc_sh,
          usem0, usem1, isem):
        cc = lax.axis_index("c")
        ss = lax.axis_index("s")
        wid = ss * NC + cc
        base0 = wid * per_w
        usems = (usem0, usem1)

        cpi = pltpu.async_copy(dst_hbm.at[wid], idx_sl, isem)
        z_v = u_v.at[0]

        @pl.loop(0, c)
        def _zrow(i):
            for j in range(h // 16):
                z_v[i, pl.ds(j * 16, 16)] = jnp.zeros((16,), jnp.float32)

        for r in range(zreps):
            pltpu.sync_copy(
                z_v, acc_sh.at[pl.ds(ss * rows_per_tile + r * c, c)]
            )
        cpi.wait()
        plsc.subcore_barrier()

        def issue_load(kk, si):
            pltpu.async_copy(
                ne_hbm.at[pl.ds(base0 + kk * c, c)], u_v.at[si], usems[si])

        def do_chunk(kk, si, issue_next):
            u_r = u_v.at[si]
            pltpu.make_async_copy(
                ne_hbm.at[pl.ds(base0, c)], u_r, usems[si]).wait()
            pltpu.sync_copy(u_r, acc_sh.at[idx_sl.at[kk]], add=True)
            if issue_next:
                issue_load(kk + 2, si)

        issue_load(0, 0)
        issue_load(1, 1)
        do_chunk(0, 0, True)
        do_chunk(1, 1, True)

        @pl.loop(1, (chunks - 5) // 2 + 1)
        def _pair(t):
            do_chunk(2 * t, 0, True)
            do_chunk(2 * t + 1, 1, True)

        do_chunk(chunks - 3, 0, True)
        do_chunk(chunks - 2, 1, False)
        do_chunk(chunks - 1, 0, False)

        plsc.subcore_barrier()
        pltpu.sync_copy(
            acc_sh.at[pl.ds(ss * rows_per_tile, rows_per_tile)],
            out_hbm.at[cc, pl.ds(ss * rows_per_tile, rows_per_tile)],
        )

    return k(ne, dst3)


def _node_body(x_ref, p0_ref, p1_ref, ux_ref, ua_ref, b0_ref, w1_ref, b1_ref,
               lng_ref, lnb_ref, out_ref):
    xb = x_ref[...]
    agg = p0_ref[...] + p1_ref[...]
    u = jnp.maximum(
        _dot(xb, ux_ref[...]) + _dot(agg, ua_ref[...]) + b0_ref[...], 0.0)
    u = _dot(u, w1_ref[...]) + b1_ref[...]
    m = jnp.mean(u, axis=1, keepdims=True)
    v = jnp.mean((u - m) ** 2, axis=1, keepdims=True)
    out_ref[...] = (
        xb + (u - m) * lax.rsqrt(v + 1e-5) * lng_ref[...] + lnb_ref[...]
    )


def _node_mlp(x, p0, p1, ux, ua, b0, w1, b1, lng, lnb, block):
    n, d = x.shape
    h = ux.shape[1]
    grid = n // block
    return pl.pallas_call(
        _node_body,
        grid=(grid,),
        in_specs=[
            pl.BlockSpec((block, d), lambda i: (i, 0)),
            pl.BlockSpec((block, d), lambda i: (i, 0)),
            pl.BlockSpec((block, d), lambda i: (i, 0)),
            pl.BlockSpec((d, h), lambda i: (0, 0)),
            pl.BlockSpec((d, h), lambda i: (0, 0)),
            pl.BlockSpec((1, h), lambda i: (0, 0)),
            pl.BlockSpec((h, d), lambda i: (0, 0)),
            pl.BlockSpec((1, d), lambda i: (0, 0)),
            pl.BlockSpec((1, d), lambda i: (0, 0)),
            pl.BlockSpec((1, d), lambda i: (0, 0)),
        ],
        out_specs=pl.BlockSpec((block, d), lambda i: (i, 0)),
        out_shape=jax.ShapeDtypeStruct((n, d), jnp.float32),
    )(x, p0, p1, ux, ua, b0, w1, b1, lng, lnb)


def kernel(x, edge_index, edge_attr, msg_W0, msg_b0, msg_W1, msg_b1,
           msg_lng, msg_lnb, upd_W0, upd_b0, upd_W1, upd_b1, upd_lng,
           upd_lnb):
    n, d = x.shape
    e = edge_index.shape[1]
    h = msg_W0.shape[1]
    chunks = e // (NW * CHUNK)
    assert chunks * NW * CHUNK == e
    src3 = edge_index[0].reshape(NW, chunks, CHUNK)
    dst3 = edge_index[1].reshape(NW, chunks, CHUNK)

    w0_i = msg_W0[:d]
    w0_j = msg_W0[d:2 * d]
    w0_e = msg_W0[2 * d:]
    b0 = msg_b0.reshape(1, h)
    b1 = msg_b1.reshape(1, d)
    lng = msg_lng.reshape(1, d)
    lnb = msg_lnb.reshape(1, d)

    pi, pj = _precompute(x, w0_i, w0_j, b0, block=2000)
    g = _gather_add(pi, pj, dst3, src3)
    new_edge_attr = _edge_mlp(g, edge_attr, w0_e, msg_W1, b1, lng, lnb,
                              block=8000)
    partials = _scatter_add(new_edge_attr, dst3, n)

    ux = upd_W0[:d]
    ua = upd_W0[d:]
    new_x = _node_mlp(
        x, partials[0], partials[1], ux, ua, upd_b0.reshape(1, h),
        upd_W1, upd_b1.reshape(1, d), upd_lng.reshape(1, d),
        upd_lnb.reshape(1, d), block=2000,
    )
    return (new_x, new_edge_attr)
